# baseline (speedup 1.0000x reference)
import jax
import jax.numpy as jnp
from jax import lax
from jax.experimental import pallas as pl
from jax.experimental.pallas import tpu as pltpu
from jax.experimental.pallas import tpu_sc as plsc

_N_ATOMS = 10000
_N_PAIRS = 320000
_N_FEAT = 128
_N_BASIS = 64
_CUTOFF = 8.0

_NC, _NS, _L = 2, 16, 16
_NW = _NC * _NS

_AQ = 320
_AG = 80
_PQ = _N_PAIRS // _NW
_PV = _PQ // _L


def _table_scale_body(w_ref, out_ref):
    w = w_ref[...]
    norm = jnp.sqrt(jnp.sum(w * w, axis=1, keepdims=True))
    scale = jnp.minimum(1.0, 1.0 / jnp.maximum(norm, 1e-12))
    out_ref[...] = w * scale


def _feat_body(nums_hbm, table_hbm, out_hbm, idx_v, rows_v, sem):
    wid = lax.axis_index("s") * _NC + lax.axis_index("c")
    base = jnp.minimum(wid * _AQ, _N_ATOMS - _AQ)
    pltpu.sync_copy(nums_hbm.at[pl.ds(base, _AQ)], idx_v)
    cps = [
        pltpu.async_copy(
            table_hbm.at[idx_v.at[pl.ds(_AG * k, _AG)]],
            rows_v.at[pl.ds(_AG * k, _AG)],
            sem,
        )
        for k in range(_AQ // _AG)
    ]
    for cp in cps:
        cp.wait()
    pltpu.sync_copy(rows_v, out_hbm.at[pl.ds(base, _AQ)])


def _pairs_body(pos_hbm, ii_hbm, jj_hbm, vec_hbm, d2_hbm,
                pos_v, ii_v, jj_v, vec_v, d2_v, sem):
    wid = lax.axis_index("s") * _NC + lax.axis_index("c")
    base = wid * _PQ
    cp_pos = pltpu.async_copy(pos_hbm, pos_v, sem)
    cp_i = pltpu.async_copy(ii_hbm.at[pl.ds(base, _PQ)], ii_v, sem)
    cp_j = pltpu.async_copy(jj_hbm.at[pl.ds(base, _PQ)], jj_v, sem)
    cp_pos.wait()
    cp_i.wait()
    cp_j.wait()

    def step(t, carry):
        off = pl.multiple_of(t * _L, _L)
        i16 = ii_v[pl.ds(off, _L)]
        j16 = jj_v[pl.ds(off, _L)]
        fi = i16 * 3
        fj = j16 * 3
        xi = plsc.load_gather(pos_v, [fi])
        yi = plsc.load_gather(pos_v, [fi + 1])
        zi = plsc.load_gather(pos_v, [fi + 2])
        xj = plsc.load_gather(pos_v, [fj])
        yj = plsc.load_gather(pos_v, [fj + 1])
        zj = plsc.load_gather(pos_v, [fj + 2])
        dx = xj - xi
        dy = yj - yi
        dz = zj - zi
        vec_v[pl.ds(off, _L)] = dx
        vec_v[pl.ds(_PQ + off, _L)] = dy
        vec_v[pl.ds(2 * _PQ + off, _L)] = dz
        d2_v[pl.ds(off, _L)] = dx * dx + dy * dy + dz * dz
        return carry

    lax.fori_loop(0, _PV, step, 0)
    pltpu.sync_copy(vec_v.at[pl.ds(0, _PQ)], vec_hbm.at[pl.ds(base, _PQ)])
    pltpu.sync_copy(vec_v.at[pl.ds(_PQ, _PQ)],
                    vec_hbm.at[pl.ds(_N_PAIRS + base, _PQ)])
    pltpu.sync_copy(vec_v.at[pl.ds(2 * _PQ, _PQ)],
                    vec_hbm.at[pl.ds(2 * _N_PAIRS + base, _PQ)])
    pltpu.sync_copy(d2_v, d2_hbm.at[pl.ds(base, _PQ)])


def _dense_body(d2_ref, c_ref, w_ref, dist_ref, cut_ref, rbf_ref):
    d2 = d2_ref[...]
    d = jnp.sqrt(d2)
    dist_ref[...] = d
    x = d * (1.0 / _CUTOFF)
    x2 = x * x
    x3 = x2 * x
    cut_ref[...] = jnp.where(x < 1.0, 1.0 - x3 * (10.0 - 15.0 * x + 6.0 * x2), 0.0)
    c = c_ref[...]
    winv = 1.0 / w_ref[...]
    drow = d.reshape(1, d.shape[0])
    t = (drow - c) * winv
    rbf_ref[...] = jnp.exp(-0.5 * t * t)


def kernel(atomic_numbers, positions, idx_i, idx_j, atom_features_weight,
           rbf_centers, rbf_widths):
    f32 = jnp.float32
    i32 = jnp.int32
    mesh = plsc.VectorSubcoreMesh(core_axis_name="c", subcore_axis_name="s",
                                  num_cores=_NC, num_subcores=_NS)

    scaled = pl.pallas_call(
        _table_scale_body,
        out_shape=jax.ShapeDtypeStruct(atom_features_weight.shape, f32),
    )(atom_features_weight)

    feat_k = pl.kernel(
        _feat_body,
        out_type=jax.ShapeDtypeStruct((_N_ATOMS, _N_FEAT), f32),
        mesh=mesh,
        compiler_params=pltpu.CompilerParams(needs_layout_passes=False),
        scratch_types=[
            pltpu.VMEM((_AQ,), i32),
            pltpu.VMEM((_AQ, _N_FEAT), f32),
            pltpu.SemaphoreType.DMA,
        ],
    )
    features = feat_k(atomic_numbers.astype(i32), scaled)

    pairs_k = pl.kernel(
        _pairs_body,
        out_type=(
            jax.ShapeDtypeStruct((_N_PAIRS * 3,), f32),
            jax.ShapeDtypeStruct((_N_PAIRS,), f32),
        ),
        mesh=mesh,
        compiler_params=pltpu.CompilerParams(needs_layout_passes=False),
        scratch_types=[
            pltpu.VMEM((_N_ATOMS * 3,), f32),
            pltpu.VMEM((_PQ,), i32),
            pltpu.VMEM((_PQ,), i32),
            pltpu.VMEM((_PQ * 3,), f32),
            pltpu.VMEM((_PQ,), f32),
            pltpu.SemaphoreType.DMA,
        ],
    )
    vecflat, d2 = pairs_k(positions.reshape(-1), idx_i.astype(i32),
                          idx_j.astype(i32))

    _B = 32768
    _G = (_N_PAIRS + _B - 1) // _B
    distances, cutoffs, rbfs_t = pl.pallas_call(
        _dense_body,
        grid=(_G,),
        in_specs=[
            pl.BlockSpec((_B,), lambda i: (i,)),
            pl.BlockSpec((_N_BASIS, 1), lambda i: (0, 0)),
            pl.BlockSpec((_N_BASIS, 1), lambda i: (0, 0)),
        ],
        out_specs=[
            pl.BlockSpec((_B,), lambda i: (i,)),
            pl.BlockSpec((_B,), lambda i: (i,)),
            pl.BlockSpec((_N_BASIS, _B), lambda i: (0, i)),
        ],
        out_shape=(
            jax.ShapeDtypeStruct((_N_PAIRS,), f32),
            jax.ShapeDtypeStruct((_N_PAIRS,), f32),
            jax.ShapeDtypeStruct((_N_BASIS, _N_PAIRS), f32),
        ),
    )(d2, rbf_centers.reshape(_N_BASIS, 1), rbf_widths.reshape(_N_BASIS, 1))

    rbfs = rbfs_t.T
    vectors = vecflat.reshape(3, _N_PAIRS).T
    return (features, distances, vectors, cutoffs, rbfs, distances)


# speedup vs baseline: 1.0081x; 1.0081x over previous
import jax
import jax.numpy as jnp
from jax import lax
from jax.experimental import pallas as pl
from jax.experimental.pallas import tpu as pltpu
from jax.experimental.pallas import tpu_sc as plsc

_N_ATOMS = 10000
_N_PAIRS = 320000
_N_FEAT = 128
_N_BASIS = 64
_CUTOFF = 8.0

_NC, _NS, _L = 2, 16, 16
_NW = _NC * _NS

_AQ = 320
_AG = 80
_PQ = _N_PAIRS // _NW
_PV = _PQ // _L


def _table_scale_body(w_ref, out_ref):
    w = w_ref[...]
    norm = jnp.sqrt(jnp.sum(w * w, axis=1, keepdims=True))
    scale = jnp.minimum(1.0, 1.0 / jnp.maximum(norm, 1e-12))
    out_ref[...] = w * scale


_AC = _AQ // 2


def _sc_body(pos_hbm, ii_hbm, jj_hbm, nums_hbm, table_hbm,
             vec_hbm, d2_hbm, feat_hbm,
             pos_v, ii_v, jj_v, vec_v, d2_v, idx_v, rows_v, sem, semf):
    wid = lax.axis_index("s") * _NC + lax.axis_index("c")
    base = wid * _PQ
    basef = jnp.minimum(wid * _AQ, _N_ATOMS - _AQ)
    pltpu.sync_copy(nums_hbm.at[pl.ds(basef, _AQ)], idx_v)
    cp_pos = pltpu.async_copy(pos_hbm, pos_v, sem)
    cp_i = pltpu.async_copy(ii_hbm.at[pl.ds(base, _PQ)], ii_v, sem)
    cp_j = pltpu.async_copy(jj_hbm.at[pl.ds(base, _PQ)], jj_v, sem)
    gA = [
        pltpu.async_copy(
            table_hbm.at[idx_v.at[pl.ds(_AG * k, _AG)]],
            rows_v.at[pl.ds(_AG * k, _AG)], semf)
        for k in range(_AC // _AG)
    ]
    cp_pos.wait()
    cp_i.wait()
    cp_j.wait()

    def step(t, carry):
        off = pl.multiple_of(t * _L, _L)
        i16 = ii_v[pl.ds(off, _L)]
        j16 = jj_v[pl.ds(off, _L)]
        fi = i16 * 3
        fj = j16 * 3
        xi = plsc.load_gather(pos_v, [fi])
        yi = plsc.load_gather(pos_v, [fi + 1])
        zi = plsc.load_gather(pos_v, [fi + 2])
        xj = plsc.load_gather(pos_v, [fj])
        yj = plsc.load_gather(pos_v, [fj + 1])
        zj = plsc.load_gather(pos_v, [fj + 2])
        dx = xj - xi
        dy = yj - yi
        dz = zj - zi
        vec_v[pl.ds(off, _L)] = dx
        vec_v[pl.ds(_PQ + off, _L)] = dy
        vec_v[pl.ds(2 * _PQ + off, _L)] = dz
        d2_v[pl.ds(off, _L)] = dx * dx + dy * dy + dz * dz
        return carry

    lax.fori_loop(0, _PV // 2, step, 0)
    for cp in gA:
        cp.wait()
    pltpu.sync_copy(rows_v, feat_hbm.at[pl.ds(basef, _AC)])
    gB = [
        pltpu.async_copy(
            table_hbm.at[idx_v.at[pl.ds(_AC + _AG * k, _AG)]],
            rows_v.at[pl.ds(_AG * k, _AG)], semf)
        for k in range(_AC // _AG)
    ]
    lax.fori_loop(_PV // 2, _PV, step, 0)
    for cp in gB:
        cp.wait()
    pltpu.sync_copy(rows_v, feat_hbm.at[pl.ds(basef + _AC, _AC)])
    pltpu.sync_copy(vec_v.at[pl.ds(0, _PQ)], vec_hbm.at[pl.ds(base, _PQ)])
    pltpu.sync_copy(vec_v.at[pl.ds(_PQ, _PQ)],
                    vec_hbm.at[pl.ds(_N_PAIRS + base, _PQ)])
    pltpu.sync_copy(vec_v.at[pl.ds(2 * _PQ, _PQ)],
                    vec_hbm.at[pl.ds(2 * _N_PAIRS + base, _PQ)])
    pltpu.sync_copy(d2_v, d2_hbm.at[pl.ds(base, _PQ)])


def _dense_body(d2_ref, c_ref, w_ref, dist_ref, cut_ref, rbf_ref):
    d2 = d2_ref[...]
    d = jnp.sqrt(d2)
    dist_ref[...] = d
    x = d * (1.0 / _CUTOFF)
    x2 = x * x
    x3 = x2 * x
    cut_ref[...] = jnp.where(x < 1.0, 1.0 - x3 * (10.0 - 15.0 * x + 6.0 * x2), 0.0)
    c = c_ref[...]
    winv = 1.0 / w_ref[...]
    drow = d.reshape(1, d.shape[0])
    t = (drow - c) * winv
    rbf_ref[...] = jnp.exp(-0.5 * t * t)


def kernel(atomic_numbers, positions, idx_i, idx_j, atom_features_weight,
           rbf_centers, rbf_widths):
    f32 = jnp.float32
    i32 = jnp.int32
    mesh = plsc.VectorSubcoreMesh(core_axis_name="c", subcore_axis_name="s",
                                  num_cores=_NC, num_subcores=_NS)

    scaled = pl.pallas_call(
        _table_scale_body,
        out_shape=jax.ShapeDtypeStruct(atom_features_weight.shape, f32),
    )(atom_features_weight)

    sc_k = pl.kernel(
        _sc_body,
        out_type=(
            jax.ShapeDtypeStruct((_N_PAIRS * 3,), f32),
            jax.ShapeDtypeStruct((_N_PAIRS,), f32),
            jax.ShapeDtypeStruct((_N_ATOMS, _N_FEAT), f32),
        ),
        mesh=mesh,
        compiler_params=pltpu.CompilerParams(needs_layout_passes=False),
        scratch_types=[
            pltpu.VMEM((_N_ATOMS * 3,), f32),
            pltpu.VMEM((_PQ,), i32),
            pltpu.VMEM((_PQ,), i32),
            pltpu.VMEM((_PQ * 3,), f32),
            pltpu.VMEM((_PQ,), f32),
            pltpu.VMEM((_AQ,), i32),
            pltpu.VMEM((_AC, _N_FEAT), f32),
            pltpu.SemaphoreType.DMA,
            pltpu.SemaphoreType.DMA,
        ],
    )
    vecflat, d2, features = sc_k(positions.reshape(-1), idx_i.astype(i32),
                                 idx_j.astype(i32), atomic_numbers.astype(i32),
                                 scaled)

    _B = 16384
    _G = (_N_PAIRS + _B - 1) // _B
    distances, cutoffs, rbfs_t = pl.pallas_call(
        _dense_body,
        grid=(_G,),
        in_specs=[
            pl.BlockSpec((_B,), lambda i: (i,)),
            pl.BlockSpec((_N_BASIS, 1), lambda i: (0, 0)),
            pl.BlockSpec((_N_BASIS, 1), lambda i: (0, 0)),
        ],
        out_specs=[
            pl.BlockSpec((_B,), lambda i: (i,)),
            pl.BlockSpec((_B,), lambda i: (i,)),
            pl.BlockSpec((_N_BASIS, _B), lambda i: (0, i)),
        ],
        out_shape=(
            jax.ShapeDtypeStruct((_N_PAIRS,), f32),
            jax.ShapeDtypeStruct((_N_PAIRS,), f32),
            jax.ShapeDtypeStruct((_N_BASIS, _N_PAIRS), f32),
        ),
    )(d2, rbf_centers.reshape(_N_BASIS, 1), rbf_widths.reshape(_N_BASIS, 1))

    rbfs = rbfs_t.T
    vectors = vecflat.reshape(3, _N_PAIRS).T
    return (features, distances, vectors, cutoffs, rbfs, distances)


# speedup vs baseline: 1.0165x; 1.0083x over previous
import jax
import jax.numpy as jnp
from jax import lax
from jax.experimental import pallas as pl
from jax.experimental.pallas import tpu as pltpu
from jax.experimental.pallas import tpu_sc as plsc

_N_ATOMS = 10000
_N_PAIRS = 320000
_N_FEAT = 128
_N_BASIS = 64
_CUTOFF = 8.0

_NC, _NS, _L = 2, 16, 16
_NW = _NC * _NS

_AQ = 320
_AG = 80
_PQ = _N_PAIRS // _NW
_PV = _PQ // _L


def _table_scale_body(w_ref, out_ref):
    w = w_ref[...]
    norm = jnp.sqrt(jnp.sum(w * w, axis=1, keepdims=True))
    scale = jnp.minimum(1.0, 1.0 / jnp.maximum(norm, 1e-12))
    out_ref[...] = w * scale


_AC = _AQ // 2


def _sc_body(pos_hbm, ii_hbm, jj_hbm, nums_hbm, table_hbm,
             vec_hbm, d2_hbm, feat_hbm,
             pos_v, ii_v, jj_v, vec_v, d2_v, idx_v, rows_v, sem, semf):
    wid = lax.axis_index("s") * _NC + lax.axis_index("c")
    base = wid * _PQ
    basef = jnp.minimum(wid * _AQ, _N_ATOMS - _AQ)
    pltpu.sync_copy(nums_hbm.at[pl.ds(basef, _AQ)], idx_v)
    cp_pos = pltpu.async_copy(pos_hbm, pos_v, sem)
    cp_i = pltpu.async_copy(ii_hbm.at[pl.ds(base, _PQ)], ii_v, sem)
    cp_j = pltpu.async_copy(jj_hbm.at[pl.ds(base, _PQ)], jj_v, sem)
    gA = [
        pltpu.async_copy(
            table_hbm.at[idx_v.at[pl.ds(_AG * k, _AG)]],
            rows_v.at[pl.ds(_AG * k, _AG)], semf)
        for k in range(_AC // _AG)
    ]
    cp_pos.wait()
    cp_i.wait()
    cp_j.wait()

    def step(t):
        off = pl.multiple_of(t * _L, _L)
        i16 = ii_v[pl.ds(off, _L)]
        j16 = jj_v[pl.ds(off, _L)]
        fi = i16 * 3
        fj = j16 * 3
        xi = plsc.load_gather(pos_v, [fi])
        yi = plsc.load_gather(pos_v, [fi + 1])
        zi = plsc.load_gather(pos_v, [fi + 2])
        xj = plsc.load_gather(pos_v, [fj])
        yj = plsc.load_gather(pos_v, [fj + 1])
        zj = plsc.load_gather(pos_v, [fj + 2])
        dx = xj - xi
        dy = yj - yi
        dz = zj - zi
        vec_v[pl.ds(off, _L)] = dx
        vec_v[pl.ds(_PQ + off, _L)] = dy
        vec_v[pl.ds(2 * _PQ + off, _L)] = dz
        d2_v[pl.ds(off, _L)] = dx * dx + dy * dy + dz * dz

    plsc.parallel_loop(0, _PV // 2, unroll=4)(step)
    for cp in gA:
        cp.wait()
    pltpu.sync_copy(rows_v, feat_hbm.at[pl.ds(basef, _AC)])
    gB = [
        pltpu.async_copy(
            table_hbm.at[idx_v.at[pl.ds(_AC + _AG * k, _AG)]],
            rows_v.at[pl.ds(_AG * k, _AG)], semf)
        for k in range(_AC // _AG)
    ]
    plsc.parallel_loop(_PV // 2, _PV, unroll=4)(step)
    for cp in gB:
        cp.wait()
    pltpu.sync_copy(rows_v, feat_hbm.at[pl.ds(basef + _AC, _AC)])
    pltpu.sync_copy(vec_v.at[pl.ds(0, _PQ)], vec_hbm.at[pl.ds(base, _PQ)])
    pltpu.sync_copy(vec_v.at[pl.ds(_PQ, _PQ)],
                    vec_hbm.at[pl.ds(_N_PAIRS + base, _PQ)])
    pltpu.sync_copy(vec_v.at[pl.ds(2 * _PQ, _PQ)],
                    vec_hbm.at[pl.ds(2 * _N_PAIRS + base, _PQ)])
    pltpu.sync_copy(d2_v, d2_hbm.at[pl.ds(base, _PQ)])


def _dense_body(d2_ref, c_ref, w_ref, dist_ref, cut_ref, rbf_ref):
    d2 = d2_ref[...]
    d = jnp.sqrt(d2)
    dist_ref[...] = d
    x = d * (1.0 / _CUTOFF)
    x2 = x * x
    x3 = x2 * x
    cut_ref[...] = jnp.where(x < 1.0, 1.0 - x3 * (10.0 - 15.0 * x + 6.0 * x2), 0.0)
    c = c_ref[...]
    winv = 1.0 / w_ref[...]
    drow = d.reshape(1, d.shape[0])
    t = (drow - c) * winv
    rbf_ref[...] = jnp.exp(-0.5 * t * t)


def kernel(atomic_numbers, positions, idx_i, idx_j, atom_features_weight,
           rbf_centers, rbf_widths):
    f32 = jnp.float32
    i32 = jnp.int32
    mesh = plsc.VectorSubcoreMesh(core_axis_name="c", subcore_axis_name="s",
                                  num_cores=_NC, num_subcores=_NS)

    scaled = pl.pallas_call(
        _table_scale_body,
        out_shape=jax.ShapeDtypeStruct(atom_features_weight.shape, f32),
    )(atom_features_weight)

    sc_k = pl.kernel(
        _sc_body,
        out_type=(
            jax.ShapeDtypeStruct((_N_PAIRS * 3,), f32),
            jax.ShapeDtypeStruct((_N_PAIRS,), f32),
            jax.ShapeDtypeStruct((_N_ATOMS, _N_FEAT), f32),
        ),
        mesh=mesh,
        compiler_params=pltpu.CompilerParams(needs_layout_passes=False),
        scratch_types=[
            pltpu.VMEM((_N_ATOMS * 3,), f32),
            pltpu.VMEM((_PQ,), i32),
            pltpu.VMEM((_PQ,), i32),
            pltpu.VMEM((_PQ * 3,), f32),
            pltpu.VMEM((_PQ,), f32),
            pltpu.VMEM((_AQ,), i32),
            pltpu.VMEM((_AC, _N_FEAT), f32),
            pltpu.SemaphoreType.DMA,
            pltpu.SemaphoreType.DMA,
        ],
    )
    vecflat, d2, features = sc_k(positions.reshape(-1), idx_i.astype(i32),
                                 idx_j.astype(i32), atomic_numbers.astype(i32),
                                 scaled)

    _B = 16384
    _G = (_N_PAIRS + _B - 1) // _B
    distances, cutoffs, rbfs_t = pl.pallas_call(
        _dense_body,
        grid=(_G,),
        in_specs=[
            pl.BlockSpec((_B,), lambda i: (i,)),
            pl.BlockSpec((_N_BASIS, 1), lambda i: (0, 0)),
            pl.BlockSpec((_N_BASIS, 1), lambda i: (0, 0)),
        ],
        out_specs=[
            pl.BlockSpec((_B,), lambda i: (i,)),
            pl.BlockSpec((_B,), lambda i: (i,)),
            pl.BlockSpec((_N_BASIS, _B), lambda i: (0, i)),
        ],
        out_shape=(
            jax.ShapeDtypeStruct((_N_PAIRS,), f32),
            jax.ShapeDtypeStruct((_N_PAIRS,), f32),
            jax.ShapeDtypeStruct((_N_BASIS, _N_PAIRS), f32),
        ),
    )(d2, rbf_centers.reshape(_N_BASIS, 1), rbf_widths.reshape(_N_BASIS, 1))

    rbfs = rbfs_t.T
    vectors = vecflat.reshape(3, _N_PAIRS).T
    return (features, distances, vectors, cutoffs, rbfs, distances)
